# Initial kernel scaffold; baseline (speedup 1.0000x reference)
import jax
import jax.numpy as jnp
from jax.experimental import pallas as pl


def kernel(enhanced_y, original_y):
    raise NotImplementedError("write your pallas kernel here")


# speedup vs baseline: 2902.1103x; 2902.1103x over previous
import functools

import jax
import jax.numpy as jnp
from jax import lax
from jax.experimental import pallas as pl
from jax.experimental.pallas import tpu as pltpu
from jax.experimental.pallas import tpu_sc as plsc

B = 16
H = 512
W = 512
NPIX = H * W
NBINS = 256
LANES = 16
CH = 16384
NCHUNK = NPIX // CH
SMOOTH = 1e-6
EPS = 1e-6


def _sc_hist_kernel(e_hbm, o_hbm, out_hbm, buf0, buf1, hist, hist256,
                    pairs, accv, shared, sem0, sem1):
    c = lax.axis_index("c")
    s = lax.axis_index("s")
    b = 8 * c + s // 2
    t = s % 2
    lanebase = lax.iota(jnp.int32, LANES) * NBINS

    zero16 = jnp.zeros((LANES,), jnp.float32)
    for i in range(LANES * NBINS // LANES):
        hist[pl.ds(i * LANES, LANES)] = zero16

    bufs = (buf0, buf1)
    sems = (sem0, sem1)

    def start(ci, buf, sem):
        off = b * NPIX + ci * CH

        @pl.when(t == 0)
        def _():
            pltpu.async_copy(e_hbm.at[pl.ds(off, CH)], buf, sem)

        @pl.when(t == 1)
        def _():
            pltpu.async_copy(o_hbm.at[pl.ds(off, CH)], buf, sem)

    start(0, bufs[0], sems[0])
    for ci in range(NCHUNK):
        if ci + 1 < NCHUNK:
            start(ci + 1, bufs[(ci + 1) % 2], sems[(ci + 1) % 2])
        buf = bufs[ci % 2]
        pltpu.make_async_copy(e_hbm.at[pl.ds(0, CH)], buf, sems[ci % 2]).wait()

        def body(i, carry, buf=buf):
            base = i * (8 * LANES)
            for u in range(8):
                x = buf[pl.ds(base + u * LANES, LANES)]
                idx = (x * 256.0).astype(jnp.int32)
                idx = jnp.minimum(jnp.maximum(idx, 0), NBINS - 1)
                w = jnp.sign(x)
                plsc.addupdate_scatter(hist, [lanebase + idx], w)
            return carry

        lax.fori_loop(0, CH // (8 * LANES), body, 0)

    for j in range(NBINS // LANES):
        acc = zero16
        for l in range(LANES):
            acc = acc + hist[pl.ds(l * NBINS + j * LANES, LANES)]
        hist256[pl.ds(j * LANES, LANES)] = acc

    pltpu.sync_copy(hist256, shared.at[s])
    plsc.subcore_barrier()

    @pl.when(s == 0)
    def _():
        pltpu.sync_copy(shared, pairs)
        acc = zero16
        for k in range(8):
            sev = zero16
            sov = zero16
            for j in range(NBINS // LANES):
                sev = sev + pairs[2 * k, pl.ds(j * LANES, LANES)]
                sov = sov + pairs[2 * k + 1, pl.ds(j * LANES, LANES)]
            ones = jnp.ones((LANES,), jnp.float32)
            re = ones / (jnp.broadcast_to(jnp.sum(sev), (LANES,)) + SMOOTH)
            ro = ones / (jnp.broadcast_to(jnp.sum(sov), (LANES,)) + SMOOTH)
            for j in range(NBINS // LANES):
                he = (pairs[2 * k, pl.ds(j * LANES, LANES)] + SMOOTH) * re
                ho = (pairs[2 * k + 1, pl.ds(j * LANES, LANES)] + SMOOTH) * ro
                d = he - ho
                acc = acc + d * d
        accv[...] = acc
        pltpu.sync_copy(accv, out_hbm.at[c])


def _sc_histogram_partials(e_flat, o_flat):
    mesh = plsc.VectorSubcoreMesh(core_axis_name="c", subcore_axis_name="s")
    kern = functools.partial(
        pl.kernel,
        out_type=jax.ShapeDtypeStruct((2, LANES), jnp.float32),
        mesh=mesh,
        compiler_params=pltpu.CompilerParams(needs_layout_passes=False),
        scratch_types=[
            pltpu.VMEM((CH,), jnp.float32),
            pltpu.VMEM((CH,), jnp.float32),
            pltpu.VMEM((LANES * NBINS,), jnp.float32),
            pltpu.VMEM((NBINS,), jnp.float32),
            pltpu.VMEM((LANES, NBINS), jnp.float32),
            pltpu.VMEM((LANES,), jnp.float32),
            pltpu.VMEM_SHARED((LANES, NBINS), jnp.float32),
            pltpu.SemaphoreType.DMA,
            pltpu.SemaphoreType.DMA,
        ],
    )(_sc_hist_kernel)
    return kern(e_flat, o_flat)


def _lap_abs_sum(a):
    zr = jnp.zeros((1, W), jnp.float32)
    zc = jnp.zeros((H, 1), jnp.float32)
    up = jnp.concatenate([zr, a[:-1, :]], axis=0)
    dn = jnp.concatenate([a[1:, :], zr], axis=0)
    lf = jnp.concatenate([zc, a[:, :-1]], axis=1)
    rt = jnp.concatenate([a[:, 1:], zc], axis=1)
    return jnp.sum(jnp.abs(up + dn + lf + rt - 4.0 * a))


def _tc_dense_kernel(e_ref, o_ref, out_ref):
    bidx = pl.program_id(0)
    a = e_ref[0]
    ao = o_ref[0]

    l1 = jnp.sum(jnp.abs(a - ao))
    lape = _lap_abs_sum(a)
    lapo = _lap_abs_sum(ao)
    dxs = jnp.sum(jnp.abs(a[1:, :] - a[:-1, :]))
    dys = jnp.sum(jnp.abs(a[:, 1:] - a[:, :-1]))

    @pl.when(bidx == 0)
    def _():
        for i in range(8):
            out_ref[i] = 0.0

    out_ref[0] += l1
    out_ref[1] += lape
    out_ref[2] += lapo
    out_ref[3] += dxs
    out_ref[4] += dys


def _tc_dense_sums(e3, o3):
    return pl.pallas_call(
        _tc_dense_kernel,
        grid=(B,),
        in_specs=[
            pl.BlockSpec((1, H, W), lambda b: (b, 0, 0)),
            pl.BlockSpec((1, H, W), lambda b: (b, 0, 0)),
        ],
        out_specs=pl.BlockSpec(memory_space=pltpu.SMEM),
        out_shape=jax.ShapeDtypeStruct((8,), jnp.float32),
    )(e3, o3)


def kernel(enhanced_y, original_y):
    e3 = enhanced_y.reshape(B, H, W)
    o3 = original_y.reshape(B, H, W)

    hist_partials = _sc_histogram_partials(
        enhanced_y.reshape(B * NPIX), original_y.reshape(B * NPIX))
    sums = _tc_dense_sums(e3, o3)

    n = float(B * NPIX)
    l1 = sums[0] / n
    ce = sums[1] / n
    co = sums[2] / n
    cont = jnp.abs(ce - co) / (co + EPS)
    nd = float(B * (H - 1) * W)
    smooth = sums[3] / nd + sums[4] / nd
    hist_loss = jnp.sum(hist_partials) / float(B * NBINS) / float(NBINS)
    return l1 + 0.1 * hist_loss + 0.1 * cont + 0.01 * smooth


# speedup vs baseline: 7009.8484x; 2.4154x over previous
import functools

import jax
import jax.numpy as jnp
from jax import lax
from jax.experimental import pallas as pl
from jax.experimental.pallas import tpu as pltpu
from jax.experimental.pallas import tpu_sc as plsc

B = 16
H = 512
W = 512
NPIX = H * W
NBINS = 256
LANES = 16
CH = 16384
NCHUNK = NPIX // CH
SMOOTH = 1e-6
EPS = 1e-6


def _sc_hist_kernel(e_hbm, o_hbm, out_hbm, buf0, buf1, hist, hist256,
                    pairs, accv, shared, sem0, sem1):
    c = lax.axis_index("c")
    s = lax.axis_index("s")
    b = 8 * c + s // 2
    t = s % 2
    lanebase = lax.iota(jnp.int32, LANES) * NBINS

    zero16 = jnp.zeros((LANES,), jnp.float32)
    for i in range(LANES * NBINS // LANES):
        hist[pl.ds(i * LANES, LANES)] = zero16

    bufs = (buf0, buf1)
    sems = (sem0, sem1)

    def start(ci, buf, sem):
        off = b * NPIX + ci * CH

        @pl.when(t == 0)
        def _():
            pltpu.async_copy(e_hbm.at[pl.ds(off, CH)], buf, sem)

        @pl.when(t == 1)
        def _():
            pltpu.async_copy(o_hbm.at[pl.ds(off, CH)], buf, sem)

    start(0, bufs[0], sems[0])
    for ci in range(NCHUNK):
        if ci + 1 < NCHUNK:
            start(ci + 1, bufs[(ci + 1) % 2], sems[(ci + 1) % 2])
        buf = bufs[ci % 2]
        pltpu.make_async_copy(e_hbm.at[pl.ds(0, CH)], buf, sems[ci % 2]).wait()

        @plsc.parallel_loop(0, CH // LANES, 1, unroll=8)
        def _(i, buf=buf):
            x = buf[pl.ds(i * LANES, LANES)]
            idx = (x * 256.0).astype(jnp.int32)
            idx = jnp.minimum(jnp.maximum(idx, 0), NBINS - 1)
            w = jnp.sign(x)
            plsc.addupdate_scatter(hist, [lanebase + idx], w)

    for j in range(NBINS // LANES):
        acc = zero16
        for l in range(LANES):
            acc = acc + hist[pl.ds(l * NBINS + j * LANES, LANES)]
        hist256[pl.ds(j * LANES, LANES)] = acc

    pltpu.sync_copy(hist256, shared.at[s])
    plsc.subcore_barrier()

    @pl.when(s == 0)
    def _():
        pltpu.sync_copy(shared, pairs)
        acc = zero16
        for k in range(8):
            sev = zero16
            sov = zero16
            for j in range(NBINS // LANES):
                sev = sev + pairs[2 * k, pl.ds(j * LANES, LANES)]
                sov = sov + pairs[2 * k + 1, pl.ds(j * LANES, LANES)]
            ones = jnp.ones((LANES,), jnp.float32)
            re = ones / (jnp.broadcast_to(jnp.sum(sev), (LANES,)) + SMOOTH)
            ro = ones / (jnp.broadcast_to(jnp.sum(sov), (LANES,)) + SMOOTH)
            for j in range(NBINS // LANES):
                he = (pairs[2 * k, pl.ds(j * LANES, LANES)] + SMOOTH) * re
                ho = (pairs[2 * k + 1, pl.ds(j * LANES, LANES)] + SMOOTH) * ro
                d = he - ho
                acc = acc + d * d
        accv[...] = acc
        pltpu.sync_copy(accv, out_hbm.at[c])


def _sc_histogram_partials(e_flat, o_flat):
    mesh = plsc.VectorSubcoreMesh(core_axis_name="c", subcore_axis_name="s")
    kern = functools.partial(
        pl.kernel,
        out_type=jax.ShapeDtypeStruct((2, LANES), jnp.float32),
        mesh=mesh,
        compiler_params=pltpu.CompilerParams(needs_layout_passes=False),
        scratch_types=[
            pltpu.VMEM((CH,), jnp.float32),
            pltpu.VMEM((CH,), jnp.float32),
            pltpu.VMEM((LANES * NBINS,), jnp.float32),
            pltpu.VMEM((NBINS,), jnp.float32),
            pltpu.VMEM((LANES, NBINS), jnp.float32),
            pltpu.VMEM((LANES,), jnp.float32),
            pltpu.VMEM_SHARED((LANES, NBINS), jnp.float32),
            pltpu.SemaphoreType.DMA,
            pltpu.SemaphoreType.DMA,
        ],
    )(_sc_hist_kernel)
    return kern(e_flat, o_flat)


def _lap_abs_sum(a):
    zr = jnp.zeros((1, W), jnp.float32)
    zc = jnp.zeros((H, 1), jnp.float32)
    up = jnp.concatenate([zr, a[:-1, :]], axis=0)
    dn = jnp.concatenate([a[1:, :], zr], axis=0)
    lf = jnp.concatenate([zc, a[:, :-1]], axis=1)
    rt = jnp.concatenate([a[:, 1:], zc], axis=1)
    return jnp.sum(jnp.abs(up + dn + lf + rt - 4.0 * a))


def _tc_dense_kernel(e_ref, o_ref, out_ref):
    bidx = pl.program_id(0)
    a = e_ref[0]
    ao = o_ref[0]

    l1 = jnp.sum(jnp.abs(a - ao))
    lape = _lap_abs_sum(a)
    lapo = _lap_abs_sum(ao)
    dxs = jnp.sum(jnp.abs(a[1:, :] - a[:-1, :]))
    dys = jnp.sum(jnp.abs(a[:, 1:] - a[:, :-1]))

    @pl.when(bidx == 0)
    def _():
        for i in range(8):
            out_ref[i] = 0.0

    out_ref[0] += l1
    out_ref[1] += lape
    out_ref[2] += lapo
    out_ref[3] += dxs
    out_ref[4] += dys


def _tc_dense_sums(e3, o3):
    return pl.pallas_call(
        _tc_dense_kernel,
        grid=(B,),
        in_specs=[
            pl.BlockSpec((1, H, W), lambda b: (b, 0, 0)),
            pl.BlockSpec((1, H, W), lambda b: (b, 0, 0)),
        ],
        out_specs=pl.BlockSpec(memory_space=pltpu.SMEM),
        out_shape=jax.ShapeDtypeStruct((8,), jnp.float32),
    )(e3, o3)


def kernel(enhanced_y, original_y):
    e3 = enhanced_y.reshape(B, H, W)
    o3 = original_y.reshape(B, H, W)

    hist_partials = _sc_histogram_partials(
        enhanced_y.reshape(B * NPIX), original_y.reshape(B * NPIX))
    sums = _tc_dense_sums(e3, o3)

    n = float(B * NPIX)
    l1 = sums[0] / n
    ce = sums[1] / n
    co = sums[2] / n
    cont = jnp.abs(ce - co) / (co + EPS)
    nd = float(B * (H - 1) * W)
    smooth = sums[3] / nd + sums[4] / nd
    hist_loss = jnp.sum(hist_partials) / float(B * NBINS) / float(NBINS)
    return l1 + 0.1 * hist_loss + 0.1 * cont + 0.01 * smooth
